# Initial kernel scaffold; baseline (speedup 1.0000x reference)
import jax
import jax.numpy as jnp
from jax.experimental import pallas as pl


def kernel(pred1, pred2, mapping1, mapping2):
    raise NotImplementedError("write your pallas kernel here")


# speedup vs baseline: 1.9299x; 1.9299x over previous
import functools

import jax
import jax.numpy as jnp
from jax import lax
from jax.experimental import pallas as pl
from jax.experimental.pallas import tpu as pltpu
from jax.experimental.pallas import tpu_sc as plsc

B = 1024
C = 256
K = 256
NW = 32
NG = 8
BPW = B // NG
CPW = C // (NW // NG)
NV = BPW // 16
KU = 16


def _sc_body(p1t_hbm, p2t_hbm, m1_hbm, m2_hbm, out_hbm,
             p1_v, p2_v, m1_v, m2_v, acc_v):
    cid = lax.axis_index("c")
    sid = lax.axis_index("s")
    wid = sid * 2 + cid
    g = lax.rem(wid, NG)
    q = lax.div(wid, NG)
    b0 = g * BPW
    c0 = q * CPW

    pltpu.sync_copy(p1t_hbm.at[:, pl.ds(b0, BPW)], p1_v)
    pltpu.sync_copy(p2t_hbm.at[:, pl.ds(b0, BPW)], p2_v)
    pltpu.sync_copy(m1_hbm.at[pl.ds(c0, CPW), :], m1_v)
    pltpu.sync_copy(m2_hbm.at[pl.ds(c0, CPW), :], m2_v)

    def c_body(ci, carry):
        def k_body(kk, accs):
            accs = list(accs)
            kbase = kk * KU
            iv1 = m1_v[ci, pl.ds(kbase, KU)]
            iv2 = m2_v[ci, pl.ds(kbase, KU)]
            for u in range(KU):
                i1 = iv1[u]
                i2 = iv2[u]
                for h in range(NV):
                    accs[h] = accs[h] + (p1_v[i1, pl.ds(h * 16, 16)]
                                         * p2_v[i2, pl.ds(h * 16, 16)])
            return tuple(accs)

        z = jnp.zeros((16,), jnp.float32)
        accs = lax.fori_loop(0, K // KU, k_body, (z,) * NV)
        for h in range(NV):
            acc_v[ci, pl.ds(h * 16, 16)] = accs[h]
        return carry

    lax.fori_loop(0, CPW, c_body, 0)
    pltpu.sync_copy(acc_v, out_hbm.at[pl.ds(c0, CPW), pl.ds(b0, BPW)])


_sc_call = functools.partial(
    pl.kernel,
    mesh=plsc.VectorSubcoreMesh(core_axis_name="c", subcore_axis_name="s"),
    out_type=jax.ShapeDtypeStruct((C, B), jnp.float32),
    scratch_types=[
        pltpu.VMEM((C, BPW), jnp.float32),
        pltpu.VMEM((C, BPW), jnp.float32),
        pltpu.VMEM((CPW, K), jnp.int32),
        pltpu.VMEM((CPW, K), jnp.int32),
        pltpu.VMEM((CPW, BPW), jnp.float32),
    ],
)(_sc_body)


def kernel(pred1, pred2, mapping1, mapping2):
    p1t = pred1.T
    p2t = pred2.T
    m1 = mapping1.astype(jnp.int32)
    m2 = mapping2.astype(jnp.int32)
    out_t = _sc_call(p1t, p2t, m1, m2)
    return out_t.T


# speedup vs baseline: 4.4269x; 2.2939x over previous
import functools

import jax
import jax.numpy as jnp
from jax import lax
from jax.experimental import pallas as pl
from jax.experimental.pallas import tpu as pltpu
from jax.experimental.pallas import tpu_sc as plsc

B = 1024
C = 256
K = 256
NW = 32
NG = 8
BPW = B // NG
WPW = BPW // 2
CPW = C // (NW // NG)
NV = WPW // 16
KU = 16

_HIMASK = -65536


def _widen(word):
    lo = lax.bitcast_convert_type(lax.shift_left(word, 16), jnp.float32)
    hi = lax.bitcast_convert_type(
        lax.bitwise_and(word, jnp.int32(_HIMASK)), jnp.float32)
    return lo, hi


def _sc_body(p1t_hbm, p2t_hbm, m1_hbm, m2_hbm, out_hbm,
             p1_v, p2_v, m1_v, m2_v, acc_v):
    cid = lax.axis_index("c")
    sid = lax.axis_index("s")
    wid = sid * 2 + cid
    g = lax.rem(wid, NG)
    q = lax.div(wid, NG)
    b0 = g * BPW
    c0 = q * CPW

    pltpu.sync_copy(p1t_hbm.at[g], p1_v)
    pltpu.sync_copy(p2t_hbm.at[g], p2_v)
    pltpu.sync_copy(m1_hbm.at[pl.ds(c0, CPW), :], m1_v)
    pltpu.sync_copy(m2_hbm.at[pl.ds(c0, CPW), :], m2_v)

    def c_body(ci, carry):
        def k_body(kk, accs):
            accs = list(accs)
            kbase = kk * KU
            iv1 = m1_v[ci, pl.ds(kbase, KU)]
            iv2 = m2_v[ci, pl.ds(kbase, KU)]
            for u in range(KU):
                i1 = iv1[u]
                i2 = iv2[u]
                for h in range(NV):
                    a_lo, a_hi = _widen(p1_v[i1, pl.ds(h * 16, 16)])
                    b_lo, b_hi = _widen(p2_v[i2, pl.ds(h * 16, 16)])
                    accs[2 * h] = accs[2 * h] + a_lo * b_lo
                    accs[2 * h + 1] = accs[2 * h + 1] + a_hi * b_hi
            return tuple(accs)

        z = jnp.zeros((16,), jnp.float32)
        accs = lax.fori_loop(0, K // KU, k_body, (z,) * (2 * NV))
        for h in range(NV):
            acc_v[ci, pl.ds(h * 32, 16)] = accs[2 * h]
            acc_v[ci, pl.ds(h * 32 + 16, 16)] = accs[2 * h + 1]
        return carry

    lax.fori_loop(0, CPW, c_body, 0)
    pltpu.sync_copy(acc_v, out_hbm.at[pl.ds(c0, CPW), pl.ds(b0, BPW)])


_sc_call = functools.partial(
    pl.kernel,
    mesh=plsc.VectorSubcoreMesh(core_axis_name="c", subcore_axis_name="s"),
    out_type=jax.ShapeDtypeStruct((C, B), jnp.float32),
    scratch_types=[
        pltpu.VMEM((C, WPW), jnp.int32),
        pltpu.VMEM((C, WPW), jnp.int32),
        pltpu.VMEM((CPW, K), jnp.int32),
        pltpu.VMEM((CPW, K), jnp.int32),
        pltpu.VMEM((CPW, BPW), jnp.float32),
    ],
)(_sc_body)


def kernel(pred1, pred2, mapping1, mapping2):
    p1t = pred1.T.astype(jnp.bfloat16)
    p2t = pred2.T.astype(jnp.bfloat16)
    p1p = (lax.bitcast_convert_type(p1t.reshape(C, B // 2, 2), jnp.int32)
           .reshape(C, NG, WPW).transpose(1, 0, 2))
    p2p = (lax.bitcast_convert_type(p2t.reshape(C, B // 2, 2), jnp.int32)
           .reshape(C, NG, WPW).transpose(1, 0, 2))
    m1 = mapping1.astype(jnp.int32)
    m2 = mapping2.astype(jnp.int32)
    out_k = _sc_call(p1p, p2p, m1, m2)
    out_t = (out_k.reshape(C, B // 32, 2, 16)
             .transpose(0, 1, 3, 2)
             .reshape(C, B))
    return out_t.T


# speedup vs baseline: 5.7137x; 1.2907x over previous
import functools

import jax
import jax.numpy as jnp
from jax import lax
from jax.experimental import pallas as pl
from jax.experimental.pallas import tpu as pltpu
from jax.experimental.pallas import tpu_sc as plsc

B = 1024
C = 256
K = 256
CT = 128
CS = C - CT
NW = 32
NG = 8
BPW = B // NG
WPW = BPW // 2
NQ = NW // NG
CPW = CS // NQ
NV = WPW // 16
KU = 16
TCU = 8

_HIMASK = -65536


def _widen(word):
    lo = lax.bitcast_convert_type(lax.shift_left(word, 16), jnp.float32)
    hi = lax.bitcast_convert_type(
        lax.bitwise_and(word, jnp.int32(_HIMASK)), jnp.float32)
    return lo, hi


def _sc_body(p1t_hbm, p2t_hbm, m1_hbm, m2_hbm, out_hbm,
             p1_v, p2_v, m1_v, m2_v, acc_v):
    cid = lax.axis_index("c")
    sid = lax.axis_index("s")
    wid = sid * 2 + cid
    g = lax.rem(wid, NG)
    q = lax.div(wid, NG)
    b0 = g * BPW
    c0 = q * CPW

    pltpu.sync_copy(p1t_hbm.at[g], p1_v)
    pltpu.sync_copy(p2t_hbm.at[g], p2_v)
    pltpu.sync_copy(m1_hbm.at[pl.ds(c0, CPW), :], m1_v)
    pltpu.sync_copy(m2_hbm.at[pl.ds(c0, CPW), :], m2_v)

    def c_body(ci, carry):
        def k_body(kk, accs):
            accs = list(accs)
            kbase = kk * KU
            iv1 = m1_v[ci, pl.ds(kbase, KU)]
            iv2 = m2_v[ci, pl.ds(kbase, KU)]
            for u in range(KU):
                i1 = iv1[u]
                i2 = iv2[u]
                for h in range(NV):
                    a_lo, a_hi = _widen(p1_v[i1, pl.ds(h * 16, 16)])
                    b_lo, b_hi = _widen(p2_v[i2, pl.ds(h * 16, 16)])
                    accs[2 * h] = accs[2 * h] + a_lo * b_lo
                    accs[2 * h + 1] = accs[2 * h + 1] + a_hi * b_hi
            return tuple(accs)

        z = jnp.zeros((16,), jnp.float32)
        accs = lax.fori_loop(0, K // KU, k_body, (z,) * (2 * NV))
        for h in range(NV):
            acc_v[ci, pl.ds(h * 32, 16)] = accs[2 * h]
            acc_v[ci, pl.ds(h * 32 + 16, 16)] = accs[2 * h + 1]
        return carry

    lax.fori_loop(0, CPW, c_body, 0)
    pltpu.sync_copy(acc_v, out_hbm.at[pl.ds(c0, CPW), pl.ds(b0, BPW)])


_sc_call = functools.partial(
    pl.kernel,
    mesh=plsc.VectorSubcoreMesh(core_axis_name="c", subcore_axis_name="s"),
    out_type=jax.ShapeDtypeStruct((CS, B), jnp.float32),
    scratch_types=[
        pltpu.VMEM((C, WPW), jnp.int32),
        pltpu.VMEM((C, WPW), jnp.int32),
        pltpu.VMEM((CPW, K), jnp.int32),
        pltpu.VMEM((CPW, K), jnp.int32),
        pltpu.VMEM((CPW, BPW), jnp.float32),
    ],
)(_sc_body)


def _tc_body(m1_ref, m2_ref, p1_ref, p2_ref, out_ref):
    def k_body(kk, acc):
        kbase = kk * TCU
        for u in range(TCU):
            i1 = m1_ref[0, 0, kbase + u]
            i2 = m2_ref[0, 0, kbase + u]
            acc = acc + p1_ref[i1] * p2_ref[i2]
        return acc

    acc = lax.fori_loop(0, K // TCU, k_body,
                        jnp.zeros((8, 128), jnp.float32))
    out_ref[0] = acc


_tc_call = pl.pallas_call(
    _tc_body,
    grid=(CT,),
    in_specs=[
        pl.BlockSpec((1, 1, K), lambda c: (c, 0, 0),
                     memory_space=pltpu.SMEM),
        pl.BlockSpec((1, 1, K), lambda c: (c, 0, 0),
                     memory_space=pltpu.SMEM),
        pl.BlockSpec((C, 8, 128), lambda c: (0, 0, 0)),
        pl.BlockSpec((C, 8, 128), lambda c: (0, 0, 0)),
    ],
    out_specs=pl.BlockSpec((1, 8, 128), lambda c: (c, 0, 0)),
    out_shape=jax.ShapeDtypeStruct((CT, 8, 128), jnp.float32),
    compiler_params=pltpu.CompilerParams(
        dimension_semantics=("arbitrary",),
    ),
)


def kernel(pred1, pred2, mapping1, mapping2):
    m1 = mapping1.astype(jnp.int32)
    m2 = mapping2.astype(jnp.int32)

    p1t = pred1.T.astype(jnp.bfloat16)
    p2t = pred2.T.astype(jnp.bfloat16)
    p1p = (lax.bitcast_convert_type(p1t.reshape(C, B // 2, 2), jnp.int32)
           .reshape(C, NG, WPW).transpose(1, 0, 2))
    p2p = (lax.bitcast_convert_type(p2t.reshape(C, B // 2, 2), jnp.int32)
           .reshape(C, NG, WPW).transpose(1, 0, 2))
    sc_out = _sc_call(p1p, p2p, m1[CT:], m2[CT:])
    sc_fixed = (sc_out.reshape(CS, B // 32, 2, 16)
                .transpose(0, 1, 3, 2)
                .reshape(CS, B))

    p1r = pred1.T.reshape(C, 8, 128)
    p2r = pred2.T.reshape(C, 8, 128)
    tc_out = _tc_call(m1[:CT].reshape(CT, 1, K),
                      m2[:CT].reshape(CT, 1, K),
                      p1r, p2r).reshape(CT, B)

    out_t = jnp.concatenate([tc_out, sc_fixed], axis=0)
    return out_t.T
